# baseline (speedup 1.0000x reference)
import functools

import jax
import jax.numpy as jnp
from jax import lax
from jax.experimental import pallas as pl
from jax.experimental.pallas import tpu as pltpu
from jax.experimental.pallas import tpu_sc as plsc

_EPS = 1e-6
_NEG = 0.2
_KNN = 16

_F32 = jnp.float32
_HI = jax.lax.Precision.HIGHEST


def _mm(a, b, dims):
    return jax.lax.dot_general(a, b, dimension_numbers=(dims, ((), ())),
                               preferred_element_type=_F32, precision=_HI)


def _knn_body(n_total, yf_all_ref, yf_blk_ref, idx_ref):
    b = pl.program_id(0)
    yf = yf_all_ref[0]
    yfb = yf_blk_ref[0]
    inner = _mm(yfb, yf, ((0,), (0,)))
    sq = jnp.sum(yf * yf, axis=0)
    sqb = jnp.sum(yfb * yfb, axis=0)
    d = 2.0 * inner - sqb[:, None] - sq[None, :]
    iota = jax.lax.broadcasted_iota(
        jnp.int32, (1, d.shape[1]), 1).astype(_F32)
    big = float(2 * n_total)
    cols = []
    dd = d
    for _ in range(_KNN):
        m = jnp.max(dd, axis=1, keepdims=True)
        hit = dd == m
        cols.append(jnp.min(jnp.where(hit, iota, big), axis=1, keepdims=True))
        dd = jnp.where(hit, -jnp.inf, dd)
    idx_ref[0] = (jnp.concatenate(cols, axis=1).astype(jnp.int32)
                  + b * n_total)


def _knn(yf):
    B, C3, N = yf.shape
    NB = 512
    return pl.pallas_call(
        functools.partial(_knn_body, N),
        grid=(B, N // NB),
        in_specs=[
            pl.BlockSpec((1, C3, N), lambda b, i: (b, 0, 0)),
            pl.BlockSpec((1, C3, NB), lambda b, i: (b, 0, i)),
        ],
        out_specs=pl.BlockSpec((1, NB, _KNN), lambda b, i: (b, i, 0)),
        out_shape=jax.ShapeDtypeStruct((B, N, _KNN), jnp.int32),
    )(yf, yf)


def _pack_bf16_pair(a, b):
    ua = jax.lax.bitcast_convert_type(a, jnp.uint32)
    ub = jax.lax.bitcast_convert_type(b, jnp.uint32)
    ra = (ua + 0x7FFF + ((ua >> 16) & 1)) & jnp.uint32(0xFFFF0000)
    rb = (ub + 0x7FFF + ((ub >> 16) & 1)) >> 16
    return jax.lax.bitcast_convert_type(ra | rb, _F32)


def _unpack_bf16_pair(w):
    u = jax.lax.bitcast_convert_type(w, jnp.uint32)
    hi = jax.lax.bitcast_convert_type(u & jnp.uint32(0xFFFF0000), _F32)
    lo = jax.lax.bitcast_convert_type(u << 16, _F32)
    return hi, lo


def _vn_leaky(p_list, d_list):
    dot = sum(p * d for p, d in zip(p_list, d_list))
    dsq = sum(d * d for d in d_list)
    coef = jnp.where(dot < 0.0, (1.0 - _NEG) * dot / (dsq + _EPS), 0.0)
    return [p - coef * d for p, d in zip(p_list, d_list)]


def _proj_body(y_ref, x_ref, wqf_ref, wqd_ref, wkf_ref, wkd_ref,
               wvf_ref, wvd_ref, t_ref, cc_ref, qx_ref):
    C = y_ref.shape[1]
    wkf, wkd = wkf_ref[...], wkd_ref[...]
    wvf, wvd = wvf_ref[...], wvd_ref[...]
    wa = jnp.concatenate([wkf[:, :C], wkd[:, :C], wvf[:, :C], wvd[:, :C]],
                         axis=0)
    wb = jnp.concatenate([wkf[:, C:] - wkf[:, :C], wkd[:, C:] - wkd[:, :C],
                          wvf[:, C:] - wvf[:, :C], wvd[:, C:] - wvd[:, :C]],
                         axis=0)
    wqf, wqd = wqf_ref[...], wqd_ref[...]

    ta, qp, qd = [], [], []
    for i in range(3):
        yt = y_ref[0, :, i, :].T
        xt = x_ref[0, :, i, :].T
        a = _mm(yt, wa, ((1,), (1,)))
        for t in range(4):
            ta.append(_pack_bf16_pair(a[:, t * C:t * C + C // 2],
                                      a[:, t * C + C // 2:(t + 1) * C]))
        cc_ref[0, i * 4 * C:(i + 1) * 4 * C, :] = _mm(yt, wb, ((1,), (1,))).T
        qp.append(_mm(xt, wqf, ((1,), (1,))))
        qd.append(_mm(xt, wqd, ((1,), (1,))))
    t_ref[0, :, :6 * C] = jnp.concatenate(ta, axis=1)

    qo = _vn_leaky(qp, qd)
    n3 = jnp.sqrt(sum(q * q for q in qo))
    nf = jnp.sqrt(jnp.sum(n3 * n3, axis=1, keepdims=True))
    scale = (n3 / jnp.maximum(n3, 1e-12)) / jnp.maximum(nf, 1e-12)
    scale = scale * (1.0 / jnp.sqrt(3.0 * C))
    for i in range(3):
        qx_ref[0, :, i, :] = (qo[i] * scale).T


def _proj(y, x, wqf, wqd, wkf, wkd, wvf, wvd):
    B, C, _, N = y.shape
    NB = 512
    wspec = lambda shp: pl.BlockSpec(shp, lambda b, i: (0, 0))
    return pl.pallas_call(
        _proj_body,
        grid=(B, N // NB),
        in_specs=[
            pl.BlockSpec((1, C, 3, NB), lambda b, i: (b, 0, 0, i)),
            pl.BlockSpec((1, C, 3, NB), lambda b, i: (b, 0, 0, i)),
            wspec((C, C)), wspec((C, C)),
            wspec((C, 2 * C)), wspec((C, 2 * C)),
            wspec((C, 2 * C)), wspec((C, 2 * C)),
        ],
        out_specs=[
            pl.BlockSpec((1, NB, 8 * C), lambda b, i: (b, i, 0)),
            pl.BlockSpec((1, 12 * C, NB), lambda b, i: (b, 0, i)),
            pl.BlockSpec((1, C, 3, NB), lambda b, i: (b, 0, 0, i)),
        ],
        out_shape=[
            jax.ShapeDtypeStruct((B, N, 8 * C), _F32),
            jax.ShapeDtypeStruct((B, 12 * C, N), _F32),
            jax.ShapeDtypeStruct((B, C, 3, N), _F32),
        ],
    )(y, x, wqf, wqd, wkf, wkd, wvf, wvd)


def _sc_gather(table, idx3):
    NW, NCH, RCH = idx3.shape
    D = table.shape[1]
    mesh = plsc.VectorSubcoreMesh(core_axis_name="c", subcore_axis_name="s")

    @functools.partial(
        pl.kernel,
        out_type=jax.ShapeDtypeStruct((NW * NCH * RCH, D), _F32),
        mesh=mesh,
        scratch_types=[
            pltpu.VMEM((NCH, RCH), jnp.int32),
            pltpu.VMEM((RCH, D), _F32),
            pltpu.VMEM((RCH, D), _F32),
            pltpu.SemaphoreType.DMA,
            pltpu.SemaphoreType.DMA,
        ],
    )
    def gather_kernel(table_hbm, idx_hbm, out_hbm, idx_v, buf0, buf1,
                      sem0, sem1):
        wid = lax.axis_index("s") * 2 + lax.axis_index("c")
        pltpu.sync_copy(idx_hbm.at[wid], idx_v)

        def pair(i, carry):
            c0 = 2 * i
            cp0 = pltpu.async_copy(table_hbm.at[idx_v.at[c0]], buf0, sem0)
            cp1 = pltpu.async_copy(table_hbm.at[idx_v.at[c0 + 1]], buf1, sem1)
            cp0.wait()
            pltpu.sync_copy(buf0,
                            out_hbm.at[pl.ds((wid * NCH + c0) * RCH, RCH)])
            cp1.wait()
            pltpu.sync_copy(buf1,
                            out_hbm.at[pl.ds((wid * NCH + c0 + 1) * RCH, RCH)])
            return carry

        lax.fori_loop(0, NCH // 2, pair, 0)

    return gather_kernel(table, idx3)


def _attend_body(g_ref, cc_ref, qx_ref, x_ref, out_ref):
    C = qx_ref.shape[1]
    NB = qx_ref.shape[3]
    K = g_ref.shape[1] // NB
    gt = g_ref[0, :, :6 * C].T

    def tbl(t, i):
        h = C // 2
        hi, lo = _unpack_bf16_pair(gt[(i * 4 + t) * h:(i * 4 + t + 1) * h, :])
        sl = jnp.concatenate([hi, lo], axis=0).reshape(C, K, NB)
        ctr = cc_ref[0, (i * 4 + t) * C:(i * 4 + t + 1) * C, :]
        return sl + ctr[:, None, :]

    pk = [tbl(0, i) for i in range(3)]
    dk = [tbl(1, i) for i in range(3)]
    pv = [tbl(2, i) for i in range(3)]
    dv = [tbl(3, i) for i in range(3)]

    ko = _vn_leaky(pk, dk)
    nsq = sum(k * k for k in ko)
    inv_nf = 1.0 / jnp.maximum(
        jnp.sqrt(jnp.sum(nsq, axis=0, keepdims=True)), 1e-12)
    scale = jnp.where(nsq >= 1e-24, inv_nf,
                      jnp.sqrt(nsq) * (1e12 * inv_nf))

    qk = sum((ko[i] * scale) * qx_ref[0, :, i, :][:, None, :] for i in range(3))
    e = jnp.exp(qk)
    att = e * (1.0 / jnp.sum(e, axis=1, keepdims=True))

    vo = _vn_leaky(pv, dv)
    for i in range(3):
        out_ref[0, :, i, :] = x_ref[0, :, i, :] + jnp.sum(att * vo[i], axis=1)


def _attend(g, cc, qx, x):
    B, C, _, N = x.shape
    NB = 128
    nblk = N // NB
    return pl.pallas_call(
        _attend_body,
        grid=(B, nblk),
        in_specs=[
            pl.BlockSpec((1, _KNN * NB, 8 * C),
                         lambda b, i, nblk=nblk: (b * nblk + i, 0, 0)),
            pl.BlockSpec((1, 12 * C, NB), lambda b, i: (b, 0, i)),
            pl.BlockSpec((1, C, 3, NB), lambda b, i: (b, 0, 0, i)),
            pl.BlockSpec((1, C, 3, NB), lambda b, i: (b, 0, 0, i)),
        ],
        out_specs=pl.BlockSpec((1, C, 3, NB), lambda b, i: (b, 0, 0, i)),
        out_shape=jax.ShapeDtypeStruct((B, C, 3, N), _F32),
    )(g, cc, qx, x)


def kernel(x, y, wq_feat, wq_dir, wk_feat, wk_dir, wv_feat, wv_dir):
    B, C, _, N = x.shape
    NB = 128
    nblk = N // NB
    yf = y.reshape(B, C * 3, N)
    tables, cc, qx = _proj(y, x, wq_feat, wq_dir,
                           wk_feat, wk_dir, wv_feat, wv_dir)

    def reorder(idx):
        return (idx.reshape(1, nblk, NB, _KNN)
                .transpose(0, 1, 3, 2)
                .reshape(32, -1, NB))

    idxs = [_knn(yf[b:b + 1]) for b in range(B)]
    gs = [_sc_gather(tables[b], reorder(idxs[b])) for b in range(B)]
    outs = [
        _attend(gs[b].reshape(nblk, _KNN * NB, 8 * C),
                cc[b:b + 1], qx[b:b + 1], x[b:b + 1])
        for b in range(B)
    ]
    return jnp.concatenate(outs, axis=0)


# speedup vs baseline: 1.1237x; 1.1237x over previous
import functools

import jax
import jax.numpy as jnp
from jax import lax
from jax.experimental import pallas as pl
from jax.experimental.pallas import tpu as pltpu
from jax.experimental.pallas import tpu_sc as plsc

_EPS = 1e-6
_NEG = 0.2
_KNN = 16

_F32 = jnp.float32
_HI = jax.lax.Precision.HIGHEST


def _mm(a, b, dims):
    return jax.lax.dot_general(a, b, dimension_numbers=(dims, ((), ())),
                               preferred_element_type=_F32, precision=_HI)


def _knn_body(n_total, yf_all_ref, yf_blk_ref, idx_ref):
    b = pl.program_id(0)
    yf = yf_all_ref[0]
    yfb = yf_blk_ref[0]
    inner = jax.lax.dot_general(
        yfb, yf, dimension_numbers=((((0,), (0,))), ((), ())),
        preferred_element_type=_F32)
    sq = jnp.sum(yf * yf, axis=0)
    sqb = jnp.sum(yfb * yfb, axis=0)
    d = 2.0 * inner - sqb[:, None] - sq[None, :]
    iota = jax.lax.broadcasted_iota(
        jnp.int32, (1, d.shape[1]), 1).astype(_F32)
    big = float(2 * n_total)
    cols = []
    dd = d
    for _ in range(_KNN):
        m = jnp.max(dd, axis=1, keepdims=True)
        hit = dd == m
        cols.append(jnp.min(jnp.where(hit, iota, big), axis=1, keepdims=True))
        dd = jnp.where(hit, -jnp.inf, dd)
    idx_ref[0] = (jnp.concatenate(cols, axis=1).astype(jnp.int32)
                  + b * n_total)


def _knn(yf):
    B, C3, N = yf.shape
    NB = 256
    return pl.pallas_call(
        functools.partial(_knn_body, N),
        grid=(B, N // NB),
        in_specs=[
            pl.BlockSpec((1, C3, N), lambda b, i: (b, 0, 0)),
            pl.BlockSpec((1, C3, NB), lambda b, i: (b, 0, i)),
        ],
        out_specs=pl.BlockSpec((1, NB, _KNN), lambda b, i: (b, i, 0)),
        out_shape=jax.ShapeDtypeStruct((B, N, _KNN), jnp.int32),
    )(yf, yf)


def _pack_bf16_pair(a, b):
    ua = jax.lax.bitcast_convert_type(a, jnp.uint32)
    ub = jax.lax.bitcast_convert_type(b, jnp.uint32)
    ra = (ua + 0x7FFF + ((ua >> 16) & 1)) & jnp.uint32(0xFFFF0000)
    rb = (ub + 0x7FFF + ((ub >> 16) & 1)) >> 16
    return jax.lax.bitcast_convert_type(ra | rb, _F32)


def _unpack_bf16_pair(w):
    u = jax.lax.bitcast_convert_type(w, jnp.uint32)
    hi = jax.lax.bitcast_convert_type(u & jnp.uint32(0xFFFF0000), _F32)
    lo = jax.lax.bitcast_convert_type(u << 16, _F32)
    return hi, lo


def _vn_leaky(p_list, d_list):
    dot = sum(p * d for p, d in zip(p_list, d_list))
    dsq = sum(d * d for d in d_list)
    coef = jnp.where(dot < 0.0, (1.0 - _NEG) * dot / (dsq + _EPS), 0.0)
    return [p - coef * d for p, d in zip(p_list, d_list)]


def _proj_body(y_ref, x_ref, wqf_ref, wqd_ref, wkf_ref, wkd_ref,
               wvf_ref, wvd_ref, t_ref, cc_ref, qx_ref):
    C = y_ref.shape[1]
    wkf, wkd = wkf_ref[...], wkd_ref[...]
    wvf, wvd = wvf_ref[...], wvd_ref[...]
    wa = jnp.concatenate([wkf[:, :C], wkd[:, :C], wvf[:, :C], wvd[:, :C]],
                         axis=0)
    wb = jnp.concatenate([wkf[:, C:] - wkf[:, :C], wkd[:, C:] - wkd[:, :C],
                          wvf[:, C:] - wvf[:, :C], wvd[:, C:] - wvd[:, :C]],
                         axis=0)
    wqf, wqd = wqf_ref[...], wqd_ref[...]

    ta, qp, qd = [], [], []
    for i in range(3):
        yt = y_ref[0, :, i, :].T
        xt = x_ref[0, :, i, :].T
        a = _mm(yt, wa, ((1,), (1,)))
        for t in range(4):
            ta.append(_pack_bf16_pair(a[:, t * C:t * C + C // 2],
                                      a[:, t * C + C // 2:(t + 1) * C]))
        cc_ref[0, i * 4 * C:(i + 1) * 4 * C, :] = _mm(yt, wb, ((1,), (1,))).T
        qp.append(_mm(xt, wqf, ((1,), (1,))))
        qd.append(_mm(xt, wqd, ((1,), (1,))))
    t_ref[0, :, :6 * C] = jnp.concatenate(ta, axis=1)

    qo = _vn_leaky(qp, qd)
    n3 = jnp.sqrt(sum(q * q for q in qo))
    nf = jnp.sqrt(jnp.sum(n3 * n3, axis=1, keepdims=True))
    scale = (n3 / jnp.maximum(n3, 1e-12)) / jnp.maximum(nf, 1e-12)
    scale = scale * (1.0 / jnp.sqrt(3.0 * C))
    for i in range(3):
        qx_ref[0, :, i, :] = (qo[i] * scale).T


def _proj(y, x, wqf, wqd, wkf, wkd, wvf, wvd):
    B, C, _, N = y.shape
    NB = 512
    wspec = lambda shp: pl.BlockSpec(shp, lambda b, i: (0, 0))
    return pl.pallas_call(
        _proj_body,
        grid=(B, N // NB),
        in_specs=[
            pl.BlockSpec((1, C, 3, NB), lambda b, i: (b, 0, 0, i)),
            pl.BlockSpec((1, C, 3, NB), lambda b, i: (b, 0, 0, i)),
            wspec((C, C)), wspec((C, C)),
            wspec((C, 2 * C)), wspec((C, 2 * C)),
            wspec((C, 2 * C)), wspec((C, 2 * C)),
        ],
        out_specs=[
            pl.BlockSpec((1, NB, 8 * C), lambda b, i: (b, i, 0)),
            pl.BlockSpec((1, 12 * C, NB), lambda b, i: (b, 0, i)),
            pl.BlockSpec((1, C, 3, NB), lambda b, i: (b, 0, 0, i)),
        ],
        out_shape=[
            jax.ShapeDtypeStruct((B, N, 8 * C), _F32),
            jax.ShapeDtypeStruct((B, 12 * C, N), _F32),
            jax.ShapeDtypeStruct((B, C, 3, N), _F32),
        ],
    )(y, x, wqf, wqd, wkf, wkd, wvf, wvd)


def _sc_gather(table, idx3):
    NW, NCH, RCH = idx3.shape
    D = table.shape[1]
    mesh = plsc.VectorSubcoreMesh(core_axis_name="c", subcore_axis_name="s")

    @functools.partial(
        pl.kernel,
        out_type=jax.ShapeDtypeStruct((NW * NCH * RCH, D), _F32),
        mesh=mesh,
        scratch_types=[
            pltpu.VMEM((NCH, RCH), jnp.int32),
            pltpu.VMEM((RCH, D), _F32),
            pltpu.VMEM((RCH, D), _F32),
            pltpu.SemaphoreType.DMA,
            pltpu.SemaphoreType.DMA,
        ],
    )
    def gather_kernel(table_hbm, idx_hbm, out_hbm, idx_v, buf0, buf1,
                      sem0, sem1):
        wid = lax.axis_index("s") * 2 + lax.axis_index("c")
        pltpu.sync_copy(idx_hbm.at[wid], idx_v)

        def pair(i, carry):
            c0 = 2 * i
            cp0 = pltpu.async_copy(table_hbm.at[idx_v.at[c0]], buf0, sem0)
            cp1 = pltpu.async_copy(table_hbm.at[idx_v.at[c0 + 1]], buf1, sem1)
            cp0.wait()
            pltpu.sync_copy(buf0,
                            out_hbm.at[pl.ds((wid * NCH + c0) * RCH, RCH)])
            cp1.wait()
            pltpu.sync_copy(buf1,
                            out_hbm.at[pl.ds((wid * NCH + c0 + 1) * RCH, RCH)])
            return carry

        lax.fori_loop(0, NCH // 2, pair, 0)

    return gather_kernel(table, idx3)


def _attend_body(g_ref, cc_ref, qx_ref, x_ref, out_ref):
    C = qx_ref.shape[1]
    NB = qx_ref.shape[3]
    K = g_ref.shape[1] // NB
    gt = g_ref[0, :, :6 * C].T

    def tbl(t, i):
        h = C // 2
        hi, lo = _unpack_bf16_pair(gt[(i * 4 + t) * h:(i * 4 + t + 1) * h, :])
        sl = jnp.concatenate([hi, lo], axis=0).reshape(C, K, NB)
        ctr = cc_ref[0, (i * 4 + t) * C:(i * 4 + t + 1) * C, :]
        return sl + ctr[:, None, :]

    pk = [tbl(0, i) for i in range(3)]
    dk = [tbl(1, i) for i in range(3)]
    pv = [tbl(2, i) for i in range(3)]
    dv = [tbl(3, i) for i in range(3)]

    ko = _vn_leaky(pk, dk)
    nsq = sum(k * k for k in ko)
    inv_nf = 1.0 / jnp.maximum(
        jnp.sqrt(jnp.sum(nsq, axis=0, keepdims=True)), 1e-12)
    scale = jnp.where(nsq >= 1e-24, inv_nf,
                      jnp.sqrt(nsq) * (1e12 * inv_nf))

    qk = sum((ko[i] * scale) * qx_ref[0, :, i, :][:, None, :] for i in range(3))
    e = jnp.exp(qk)
    att = e * (1.0 / jnp.sum(e, axis=1, keepdims=True))

    vo = _vn_leaky(pv, dv)
    for i in range(3):
        out_ref[0, :, i, :] = x_ref[0, :, i, :] + jnp.sum(att * vo[i], axis=1)


def _attend(g, cc, qx, x):
    B, C, _, N = x.shape
    NB = 128
    nblk = N // NB
    return pl.pallas_call(
        _attend_body,
        grid=(B, nblk),
        in_specs=[
            pl.BlockSpec((1, _KNN * NB, 8 * C),
                         lambda b, i, nblk=nblk: (b * nblk + i, 0, 0)),
            pl.BlockSpec((1, 12 * C, NB), lambda b, i: (b, 0, i)),
            pl.BlockSpec((1, C, 3, NB), lambda b, i: (b, 0, 0, i)),
            pl.BlockSpec((1, C, 3, NB), lambda b, i: (b, 0, 0, i)),
        ],
        out_specs=pl.BlockSpec((1, C, 3, NB), lambda b, i: (b, 0, 0, i)),
        out_shape=jax.ShapeDtypeStruct((B, C, 3, N), _F32),
    )(g, cc, qx, x)


def kernel(x, y, wq_feat, wq_dir, wk_feat, wk_dir, wv_feat, wv_dir):
    B, C, _, N = x.shape
    NB = 128
    nblk = N // NB
    yf = y.reshape(B, C * 3, N)
    tables, cc, qx = _proj(y, x, wq_feat, wq_dir,
                           wk_feat, wk_dir, wv_feat, wv_dir)

    def reorder(idx):
        return (idx.reshape(1, nblk, NB, _KNN)
                .transpose(0, 1, 3, 2)
                .reshape(32, -1, NB))

    idxs = [_knn(yf[b:b + 1]) for b in range(B)]
    gs = [_sc_gather(tables[b], reorder(idxs[b])) for b in range(B)]
    outs = [
        _attend(gs[b].reshape(nblk, _KNN * NB, 8 * C),
                cc[b:b + 1], qx[b:b + 1], x[b:b + 1])
        for b in range(B)
    ]
    return jnp.concatenate(outs, axis=0)


# speedup vs baseline: 1.1488x; 1.0224x over previous
import functools

import jax
import jax.numpy as jnp
from jax import lax
from jax.experimental import pallas as pl
from jax.experimental.pallas import tpu as pltpu
from jax.experimental.pallas import tpu_sc as plsc

_EPS = 1e-6
_NEG = 0.2
_KNN = 16

_F32 = jnp.float32


def _mm(a, b, dims):
    return jax.lax.dot_general(a, b, dimension_numbers=(dims, ((), ())),
                               preferred_element_type=_F32)


def _knn_body(n_total, yf_all_ref, yf_blk_ref, idx_ref):
    b = pl.program_id(0)
    yf = yf_all_ref[0]
    yfb = yf_blk_ref[0]
    inner = jax.lax.dot_general(
        yfb, yf, dimension_numbers=((((0,), (0,))), ((), ())),
        preferred_element_type=_F32)
    sq = jnp.sum(yf * yf, axis=0)
    sqb = jnp.sum(yfb * yfb, axis=0)
    d = 2.0 * inner - sqb[:, None] - sq[None, :]
    iota = jax.lax.broadcasted_iota(
        jnp.int32, (1, d.shape[1]), 1).astype(_F32)
    big = float(2 * n_total)
    cols = []
    dd = d
    for _ in range(_KNN):
        m = jnp.max(dd, axis=1, keepdims=True)
        hit = dd == m
        cols.append(jnp.min(jnp.where(hit, iota, big), axis=1, keepdims=True))
        dd = jnp.where(hit, -jnp.inf, dd)
    idx_ref[0] = (jnp.concatenate(cols, axis=1).astype(jnp.int32)
                  + b * n_total)


def _knn(yf):
    B, C3, N = yf.shape
    NB = 256
    return pl.pallas_call(
        functools.partial(_knn_body, N),
        grid=(B, N // NB),
        in_specs=[
            pl.BlockSpec((1, C3, N), lambda b, i: (b, 0, 0)),
            pl.BlockSpec((1, C3, NB), lambda b, i: (b, 0, i)),
        ],
        out_specs=pl.BlockSpec((1, NB, _KNN), lambda b, i: (b, i, 0)),
        out_shape=jax.ShapeDtypeStruct((B, N, _KNN), jnp.int32),
    )(yf, yf)


def _pack_bf16_pair(a, b):
    ua = jax.lax.bitcast_convert_type(a, jnp.uint32)
    ub = jax.lax.bitcast_convert_type(b, jnp.uint32)
    ra = (ua + 0x7FFF + ((ua >> 16) & 1)) & jnp.uint32(0xFFFF0000)
    rb = (ub + 0x7FFF + ((ub >> 16) & 1)) >> 16
    return jax.lax.bitcast_convert_type(ra | rb, _F32)


def _unpack_bf16_pair(w):
    u = jax.lax.bitcast_convert_type(w, jnp.uint32)
    hi = jax.lax.bitcast_convert_type(u & jnp.uint32(0xFFFF0000), _F32)
    lo = jax.lax.bitcast_convert_type(u << 16, _F32)
    return hi, lo


def _vn_leaky(p_list, d_list):
    dot = sum(p * d for p, d in zip(p_list, d_list))
    dsq = sum(d * d for d in d_list)
    coef = jnp.where(dot < 0.0, (1.0 - _NEG) * dot / (dsq + _EPS), 0.0)
    return [p - coef * d for p, d in zip(p_list, d_list)]


def _proj_body(y_ref, x_ref, wqf_ref, wqd_ref, wkf_ref, wkd_ref,
               wvf_ref, wvd_ref, t_ref, cc_ref, qx_ref):
    C = y_ref.shape[1]
    wkf, wkd = wkf_ref[...], wkd_ref[...]
    wvf, wvd = wvf_ref[...], wvd_ref[...]
    wa = jnp.concatenate([wkf[:, :C], wkd[:, :C], wvf[:, :C], wvd[:, :C]],
                         axis=0)
    wb = jnp.concatenate([wkf[:, C:] - wkf[:, :C], wkd[:, C:] - wkd[:, :C],
                          wvf[:, C:] - wvf[:, :C], wvd[:, C:] - wvd[:, :C]],
                         axis=0)
    wqf, wqd = wqf_ref[...], wqd_ref[...]

    ta, qp, qd = [], [], []
    for i in range(3):
        yt = y_ref[0, :, i, :].T
        xt = x_ref[0, :, i, :].T
        a = _mm(yt, wa, ((1,), (1,)))
        for t in range(4):
            ta.append(_pack_bf16_pair(a[:, t * C:t * C + C // 2],
                                      a[:, t * C + C // 2:(t + 1) * C]))
        cc_ref[0, i * 4 * C:(i + 1) * 4 * C, :] = _mm(yt, wb, ((1,), (1,))).T
        qp.append(_mm(xt, wqf, ((1,), (1,))))
        qd.append(_mm(xt, wqd, ((1,), (1,))))
    t_ref[0, :, :6 * C] = jnp.concatenate(ta, axis=1)

    qo = _vn_leaky(qp, qd)
    n3 = jnp.sqrt(sum(q * q for q in qo))
    nf = jnp.sqrt(jnp.sum(n3 * n3, axis=1, keepdims=True))
    scale = (n3 / jnp.maximum(n3, 1e-12)) / jnp.maximum(nf, 1e-12)
    scale = scale * (1.0 / jnp.sqrt(3.0 * C))
    for i in range(3):
        qx_ref[0, :, i, :] = (qo[i] * scale).T


def _proj(y, x, wqf, wqd, wkf, wkd, wvf, wvd):
    B, C, _, N = y.shape
    NB = 512
    wspec = lambda shp: pl.BlockSpec(shp, lambda b, i: (0, 0))
    return pl.pallas_call(
        _proj_body,
        grid=(B, N // NB),
        in_specs=[
            pl.BlockSpec((1, C, 3, NB), lambda b, i: (b, 0, 0, i)),
            pl.BlockSpec((1, C, 3, NB), lambda b, i: (b, 0, 0, i)),
            wspec((C, C)), wspec((C, C)),
            wspec((C, 2 * C)), wspec((C, 2 * C)),
            wspec((C, 2 * C)), wspec((C, 2 * C)),
        ],
        out_specs=[
            pl.BlockSpec((1, NB, 8 * C), lambda b, i: (b, i, 0)),
            pl.BlockSpec((1, 12 * C, NB), lambda b, i: (b, 0, i)),
            pl.BlockSpec((1, C, 3, NB), lambda b, i: (b, 0, 0, i)),
        ],
        out_shape=[
            jax.ShapeDtypeStruct((B, N, 8 * C), _F32),
            jax.ShapeDtypeStruct((B, 12 * C, N), _F32),
            jax.ShapeDtypeStruct((B, C, 3, N), _F32),
        ],
    )(y, x, wqf, wqd, wkf, wkd, wvf, wvd)


def _sc_gather(table, idx3):
    NW, NCH, RCH = idx3.shape
    D = table.shape[1]
    mesh = plsc.VectorSubcoreMesh(core_axis_name="c", subcore_axis_name="s")

    @functools.partial(
        pl.kernel,
        out_type=jax.ShapeDtypeStruct((NW * NCH * RCH, D), _F32),
        mesh=mesh,
        scratch_types=[
            pltpu.VMEM((NCH, RCH), jnp.int32),
            pltpu.VMEM((RCH, D), _F32),
            pltpu.VMEM((RCH, D), _F32),
            pltpu.SemaphoreType.DMA,
            pltpu.SemaphoreType.DMA,
        ],
    )
    def gather_kernel(table_hbm, idx_hbm, out_hbm, idx_v, buf0, buf1,
                      sem0, sem1):
        wid = lax.axis_index("s") * 2 + lax.axis_index("c")
        pltpu.sync_copy(idx_hbm.at[wid], idx_v)

        def pair(i, carry):
            c0 = 2 * i
            cp0 = pltpu.async_copy(table_hbm.at[idx_v.at[c0]], buf0, sem0)
            cp1 = pltpu.async_copy(table_hbm.at[idx_v.at[c0 + 1]], buf1, sem1)
            cp0.wait()
            pltpu.sync_copy(buf0,
                            out_hbm.at[pl.ds((wid * NCH + c0) * RCH, RCH)])
            cp1.wait()
            pltpu.sync_copy(buf1,
                            out_hbm.at[pl.ds((wid * NCH + c0 + 1) * RCH, RCH)])
            return carry

        lax.fori_loop(0, NCH // 2, pair, 0)

    return gather_kernel(table, idx3)


def _attend_body(g_ref, cc_ref, qx_ref, x_ref, out_ref):
    C = qx_ref.shape[1]
    NB = qx_ref.shape[3]
    K = g_ref.shape[1] // NB
    gt = g_ref[0, :, :6 * C].T

    def tbl(t, i):
        h = C // 2
        hi, lo = _unpack_bf16_pair(gt[(i * 4 + t) * h:(i * 4 + t + 1) * h, :])
        sl = jnp.concatenate([hi, lo], axis=0).reshape(C, K, NB)
        ctr = cc_ref[0, (i * 4 + t) * C:(i * 4 + t + 1) * C, :]
        return sl + ctr[:, None, :]

    pk = [tbl(0, i) for i in range(3)]
    dk = [tbl(1, i) for i in range(3)]
    pv = [tbl(2, i) for i in range(3)]
    dv = [tbl(3, i) for i in range(3)]

    ko = _vn_leaky(pk, dk)
    nsq = sum(k * k for k in ko)
    inv_nf = 1.0 / jnp.maximum(
        jnp.sqrt(jnp.sum(nsq, axis=0, keepdims=True)), 1e-12)
    scale = jnp.where(nsq >= 1e-24, inv_nf,
                      jnp.sqrt(nsq) * (1e12 * inv_nf))

    qk = sum((ko[i] * scale) * qx_ref[0, :, i, :][:, None, :] for i in range(3))
    e = jnp.exp(qk)
    att = e * (1.0 / jnp.sum(e, axis=1, keepdims=True))

    vo = _vn_leaky(pv, dv)
    for i in range(3):
        out_ref[0, :, i, :] = x_ref[0, :, i, :] + jnp.sum(att * vo[i], axis=1)


def _attend(g, cc, qx, x):
    B, C, _, N = x.shape
    NB = 128
    nblk = N // NB
    return pl.pallas_call(
        _attend_body,
        grid=(B, nblk),
        in_specs=[
            pl.BlockSpec((1, _KNN * NB, 8 * C),
                         lambda b, i, nblk=nblk: (b * nblk + i, 0, 0)),
            pl.BlockSpec((1, 12 * C, NB), lambda b, i: (b, 0, i)),
            pl.BlockSpec((1, C, 3, NB), lambda b, i: (b, 0, 0, i)),
            pl.BlockSpec((1, C, 3, NB), lambda b, i: (b, 0, 0, i)),
        ],
        out_specs=pl.BlockSpec((1, C, 3, NB), lambda b, i: (b, 0, 0, i)),
        out_shape=jax.ShapeDtypeStruct((B, C, 3, N), _F32),
    )(g, cc, qx, x)


def kernel(x, y, wq_feat, wq_dir, wk_feat, wk_dir, wv_feat, wv_dir):
    B, C, _, N = x.shape
    NB = 128
    nblk = N // NB
    yf = y.reshape(B, C * 3, N)
    tables, cc, qx = _proj(y, x, wq_feat, wq_dir,
                           wk_feat, wk_dir, wv_feat, wv_dir)

    def reorder(idx):
        return (idx.reshape(1, nblk, NB, _KNN)
                .transpose(0, 1, 3, 2)
                .reshape(32, -1, NB))

    idxs = [_knn(yf[b:b + 1]) for b in range(B)]
    gs = [_sc_gather(tables[b], reorder(idxs[b])) for b in range(B)]
    outs = [
        _attend(gs[b].reshape(nblk, _KNN * NB, 8 * C),
                cc[b:b + 1], qx[b:b + 1], x[b:b + 1])
        for b in range(B)
    ]
    return jnp.concatenate(outs, axis=0)
